# baseline (speedup 1.0000x reference)
import functools

import jax
import jax.numpy as jnp
from jax import lax
from jax.experimental import pallas as pl
from jax.experimental.pallas import tpu as pltpu
from jax.experimental.pallas import tpu_sc as plsc

NC = 2
NS = 16
LN = 16

N = 100000
E = 3200000
G = 128

N_PAD = 100096
STRIPE = N_PAD // NS

ECH = 2000
BLK = 128

_mesh = plsc.VectorSubcoreMesh(core_axis_name="c", subcore_axis_name="s")
_sc_params = pltpu.CompilerParams(needs_layout_passes=False,
                                  use_tc_tiling_on_sc=False)


def _f32(x):
    return jnp.float32(x)


def _deg(dst):
    eslice = E // (NC * NS)
    nch = eslice // ECH

    @functools.partial(
        pl.kernel,
        out_type=jax.ShapeDtypeStruct((NC * NS * N_PAD,), jnp.float32),
        mesh=_mesh,
        compiler_params=_sc_params,
        scratch_types=[
            pltpu.VMEM((N_PAD,), jnp.float32),
            pltpu.VMEM((ECH,), jnp.int32),
        ],
    )
    def k(dst_hbm, out_hbm, dcnt, stage):
        c = lax.axis_index("c")
        s = lax.axis_index("s")
        w = c * NS + s
        zv = jnp.zeros((LN,), jnp.float32)
        ones = jnp.ones((LN,), jnp.float32)

        def zero_body(i, _):
            dcnt[pl.ds(i * LN, LN)] = zv
            return 0
        lax.fori_loop(0, N_PAD // LN, zero_body, 0)

        def chunk(ci, _):
            pltpu.sync_copy(dst_hbm.at[pl.ds(w * eslice + ci * ECH, ECH)], stage)

            def grp(g, _):
                dv = stage[pl.ds(g * LN, LN)]
                plsc.addupdate_scatter(dcnt, [dv], ones)
                return 0
            lax.fori_loop(0, ECH // LN, grp, 0)
            return 0
        lax.fori_loop(0, nch, chunk, 0)

        pltpu.sync_copy(dcnt, out_hbm.at[pl.ds(w * N_PAD, N_PAD)])

    return k(dst)


def _prop(srcE, dstE, y, F, C, RB, splits=1, loff=0):
    eslice = E // NS
    nch = eslice // ECH
    PADM = RB * BLK
    cap_rows = (ECH + PADM - 1) // PADM * RB + 1
    ZB = 2048
    czb = ZB * ((C + 8 + ZB - 1) // ZB)
    nzb = czb // ZB
    nwb = C // BLK
    wtail = C - nwb * BLK
    NOUT = N // splits

    zrows = jnp.zeros((ZB, F), jnp.float32)

    @functools.partial(
        pl.kernel,
        out_type=jax.ShapeDtypeStruct((NOUT, F), jnp.float32),
        mesh=_mesh,
        compiler_params=_sc_params,
        scratch_types=[
            pltpu.VMEM((2 * ECH,), jnp.int32),
            pltpu.VMEM((2 * ECH,), jnp.int32),
            pltpu.VMEM((2 * cap_rows, BLK), jnp.int32),
            pltpu.VMEM((2 * cap_rows, BLK), jnp.int32),
            pltpu.VMEM((2 * RB, BLK, F), jnp.float32),
            pltpu.VMEM_SHARED((czb, F), jnp.float32),
            pltpu.SemaphoreType.DMA((2 * RB,)),
            pltpu.SemaphoreType.DMA((2 * RB,)),
            pltpu.SemaphoreType.DMA((2,)),
            pltpu.SemaphoreType.DMA((2,)),
        ],
    )
    def k(src_hbm, dst_hbm, y_hbm, z_hbm, out_hbm,
          ssrc, sdst, isrc, idst, rowbuf, acc, gsem, ssem, stsrc, stdst):
        c = lax.axis_index("c")
        s = lax.axis_index("s")
        iota = lax.iota(jnp.int32, LN)
        trash = jnp.full((LN,), C, jnp.int32)
        zsrc = jnp.zeros((LN,), jnp.int32)
        base = (c * splits + loff) * C
        obase = c * C
        ebase = s * eslice

        def zblk(j, _):
            pltpu.sync_copy(z_hbm, acc.at[pl.ds((s + j * NS) * ZB, ZB), :])
            return 0
        lax.fori_loop(0, (nzb - s + NS - 1) // NS, zblk, 0)
        plsc.subcore_barrier()

        def stage(ci, p):
            off = ebase + ci * ECH
            pltpu.async_copy(src_hbm.at[pl.ds(off, ECH)],
                             ssrc.at[pl.ds(p * ECH, ECH)], stsrc.at[p])
            pltpu.async_copy(dst_hbm.at[pl.ds(off, ECH)],
                             sdst.at[pl.ds(p * ECH, ECH)], stdst.at[p])

        def stage_wait(ci, p):
            off = ebase + ci * ECH
            pltpu.make_async_copy(src_hbm.at[pl.ds(off, ECH)],
                                  ssrc.at[pl.ds(p * ECH, ECH)],
                                  stsrc.at[p]).wait()
            pltpu.make_async_copy(dst_hbm.at[pl.ds(off, ECH)],
                                  sdst.at[pl.ds(p * ECH, ECH)],
                                  stdst.at[p]).wait()

        def swait(slot):
            pltpu.make_async_copy(rowbuf.at[slot], acc.at[pl.ds(0, BLK), :],
                                  ssem.at[slot]).wait()

        stage(0, 0)

        basev = jnp.full((LN,), base, jnp.int32)
        climv = jnp.full((LN,), base + C, jnp.int32)
        c7 = jnp.full((LN,), 7, jnp.int32)
        c127 = jnp.full((LN,), 127, jnp.int32)
        rbase_v = [jnp.full((LN,), p * cap_rows, jnp.int32) for p in (0, 1)]

        def one_chunk(ci, p, outst_p):
            stage_wait(ci, p)
            @pl.when(ci + 1 < nch)
            def _():
                stage(ci + 1, 1 - p)

            for b in range(RB):
                @pl.when(b >= RB - outst_p)
                def _():
                    swait(p * RB + b)

            def grp(g, cnt):
                dv = sdst[pl.ds(p * ECH + g * LN, LN)]
                sv = ssrc[pl.ds(p * ECH + g * LN, LN)]
                m = (dv >= basev) & (dv < climv)
                mi = m.astype(jnp.int32)
                csum = plsc.cumsum(mi)
                pos = (jnp.full((LN,), cnt, jnp.int32) + csum) - mi
                row = rbase_v[p] + lax.shift_right_logical(pos, c7)
                col = pos & c127
                plsc.store_scatter(isrc, [row, col], sv, mask=m)
                plsc.store_scatter(idst, [row, col], dv - basev, mask=m)
                return cnt + jnp.sum(mi)
            cnt = lax.fori_loop(0, ECH // LN, grp, jnp.int32(0))

            npad = (PADM - (cnt & (PADM - 1))) & (PADM - 1)

            def padk(kk, _):
                pos = jnp.full((LN,), cnt + kk * LN, jnp.int32) + iota
                row = rbase_v[p] + lax.shift_right_logical(pos, c7)
                col = pos & c127
                plsc.store_scatter(isrc, [row, col], zsrc)
                plsc.store_scatter(idst, [row, col], trash)
                return 0
            lax.fori_loop(0, lax.shift_right_logical(npad + 15, 4), padk, 0)
            nblk = lax.shift_right_logical(cnt + npad, 7)

            def wave(gw, _):
                for b in range(RB):
                    j = gw * RB + b
                    @pl.when(j < nblk)
                    def _():
                        @pl.when(gw > 0)
                        def _():
                            swait(p * RB + b)
                        pltpu.async_copy(y_hbm.at[isrc.at[p * cap_rows + j]],
                                         rowbuf.at[p * RB + b], gsem.at[p * RB + b])
                for b in range(RB):
                    j = gw * RB + b
                    @pl.when(j < nblk)
                    def _():
                        pltpu.make_async_copy(
                            y_hbm.at[isrc.at[p * cap_rows + j]],
                            rowbuf.at[p * RB + b], gsem.at[p * RB + b]).wait()
                        pltpu.async_copy(rowbuf.at[p * RB + b],
                                         acc.at[idst.at[p * cap_rows + j]],
                                         ssem.at[p * RB + b], add=True)
                return 0
            lax.fori_loop(0, nblk // RB, wave, 0)
            return jnp.where(nblk > 0, jnp.int32(RB), jnp.int32(0))

        def pair(cp, carry):
            o0, o1 = carry
            o0 = one_chunk(2 * cp, 0, o0)
            o1 = one_chunk(2 * cp + 1, 1, o1)
            return (o0, o1)

        o0f, o1f = lax.fori_loop(0, nch // 2, pair,
                                 (jnp.int32(0), jnp.int32(0)))
        for p, of in ((0, o0f), (1, o1f)):
            for b in range(RB):
                @pl.when(b >= RB - of)
                def _():
                    swait(p * RB + b)

        plsc.subcore_barrier()

        def wblk(j, _):
            r = (s + j * NS) * BLK
            pltpu.sync_copy(acc.at[pl.ds(r, BLK), :],
                            out_hbm.at[pl.ds(obase + r, BLK), :])
            return 0
        lax.fori_loop(0, (nwb - s + NS - 1) // NS, wblk, 0)
        if wtail:
            @pl.when(s == 0)
            def _():
                pltpu.sync_copy(
                    acc.at[pl.ds(nwb * BLK, wtail), :],
                    out_hbm.at[pl.ds(obase + nwb * BLK, wtail), :])
        plsc.subcore_barrier()

    return k(srcE, dstE, y, zrows)


def _pool(h3flat, batchp):
    F = 64
    rslice = N // (NC * NS)
    PCH = 625
    BW = 648
    nch = rslice // PCH
    NG = (PCH + LN - 1) // LN
    FL = F // LN

    @functools.partial(
        pl.kernel,
        out_type=[
            jax.ShapeDtypeStruct((NC * NS * G * F,), jnp.float32),
            jax.ShapeDtypeStruct((NC * NS * G * F,), jnp.float32),
        ],
        mesh=_mesh,
        compiler_params=_sc_params,
        scratch_types=[
            pltpu.VMEM((PCH * F,), jnp.float32),
            pltpu.VMEM((BW,), jnp.int32),
            pltpu.VMEM((G * F,), jnp.float32),
            pltpu.VMEM((G * F,), jnp.float32),
        ],
    )
    def k(h_hbm, b_hbm, omax_hbm, osum_hbm, rows, bvm, pmax, psum):
        c = lax.axis_index("c")
        s = lax.axis_index("s")
        w = c * NS + s
        rbase = w * rslice

        ninf = jnp.full((LN,), -jnp.inf, jnp.float32)
        zv = jnp.zeros((LN,), jnp.float32)
        zi = jnp.zeros((LN,), jnp.int32)
        lane = lax.iota(jnp.int32, LN)

        def init_b(i, _):
            pmax[pl.ds(i * LN, LN)] = ninf
            psum[pl.ds(i * LN, LN)] = zv
            return 0
        lax.fori_loop(0, G * F // LN, init_b, 0)

        def chunk(ci, _):
            start = rbase + ci * PCH
            astart = (start // 8) * 8
            d = start - astart
            pltpu.sync_copy(h_hbm.at[pl.ds(start * F, PCH * F)], rows)
            pltpu.sync_copy(b_hbm.at[pl.ds(astart, BW)], bvm)

            def grpf(g, _):
                bvec = bvm[pl.ds(d + g * LN, LN)]
                for i in range(LN):
                    @pl.when(g * LN + i < PCH)
                    def _():
                        sel = lane == jnp.full((LN,), i, jnp.int32)
                        b_i = jnp.sum(jnp.where(sel, bvec, zi))
                        a = b_i * F
                        r = (g * LN + i) * F
                        for j in range(FL):
                            rv = rows[pl.ds(r + j * LN, LN)]
                            pmax[pl.ds(a + j * LN, LN)] = jnp.maximum(
                                pmax[pl.ds(a + j * LN, LN)], rv)
                            psum[pl.ds(a + j * LN, LN)] = (
                                psum[pl.ds(a + j * LN, LN)] + rv)
                return 0
            lax.fori_loop(0, NG, grpf, 0)
            return 0
        lax.fori_loop(0, nch, chunk, 0)

        pltpu.sync_copy(pmax, omax_hbm.at[pl.ds(w * G * F, G * F)])
        pltpu.sync_copy(psum, osum_hbm.at[pl.ds(w * G * F, G * F)])

    return k(h3flat, batchp)


_RB_TC = 4000
_GRID = N // _RB_TC


def _stage_a(degp, xp):
    def body(dp_ref, xp_ref, dinv_ref, y0_ref):
        d = jnp.sum(dp_ref[...], axis=1, keepdims=True)
        dv = lax.rsqrt(d + 1.0)
        dinv_ref[...] = dv
        y0_ref[...] = xp_ref[...] * dv

    return pl.pallas_call(
        body,
        grid=(_GRID,),
        in_specs=[
            pl.BlockSpec((_RB_TC, NC * NS), lambda i: (i, 0)),
            pl.BlockSpec((_RB_TC, 8), lambda i: (i, 0)),
        ],
        out_specs=[
            pl.BlockSpec((_RB_TC, 1), lambda i: (i, 0)),
            pl.BlockSpec((_RB_TC, 8), lambda i: (i, 0)),
        ],
        out_shape=[
            jax.ShapeDtypeStruct((N, 1), jnp.float32),
            jax.ShapeDtypeStruct((N, 8), jnp.float32),
        ],
    )(degp, xp)


def _stage_b3(acc_a, acc_b, y_a, y_b, dinv, Wa, Wb, b):
    def body(aa_ref, ab_ref, ya_ref, yb_ref, dinv_ref, wa_ref, wb_ref, b_ref, o_ref):
        dv = dinv_ref[...]
        pa = (aa_ref[...] + ya_ref[...]) * dv
        pb = (ab_ref[...] + yb_ref[...]) * dv
        h = (jnp.dot(pa, wa_ref[...], preferred_element_type=jnp.float32)
             + jnp.dot(pb, wb_ref[...], preferred_element_type=jnp.float32)
             + b_ref[...])
        o_ref[...] = h

    return pl.pallas_call(
        body,
        grid=(_GRID,),
        in_specs=[
            pl.BlockSpec((_RB_TC, 32), lambda i: (i, 0)),
            pl.BlockSpec((_RB_TC, 32), lambda i: (i, 0)),
            pl.BlockSpec((_RB_TC, 32), lambda i: (i, 0)),
            pl.BlockSpec((_RB_TC, 32), lambda i: (i, 0)),
            pl.BlockSpec((_RB_TC, 1), lambda i: (i, 0)),
            pl.BlockSpec((32, 64), lambda i: (0, 0)),
            pl.BlockSpec((32, 64), lambda i: (0, 0)),
            pl.BlockSpec((1, 64), lambda i: (0, 0)),
        ],
        out_specs=pl.BlockSpec((_RB_TC, 64), lambda i: (i, 0)),
        out_shape=jax.ShapeDtypeStruct((N, 64), jnp.float32),
    )(acc_a, acc_b, y_a, y_b, dinv, Wa, Wb, b.reshape(1, 64))


def _stage_b(acc, y, dinv, W, b, relu, scale):
    fi, fo = W.shape

    def body(acc_ref, y_ref, dinv_ref, w_ref, b_ref, o_ref):
        p = (acc_ref[...] + y_ref[...]) * dinv_ref[...]
        h = jnp.dot(p, w_ref[...], preferred_element_type=jnp.float32) + b_ref[...]
        if relu:
            h = jnp.maximum(h, 0.0)
        if scale:
            h = h * dinv_ref[...]
        o_ref[...] = h

    return pl.pallas_call(
        body,
        grid=(_GRID,),
        in_specs=[
            pl.BlockSpec((_RB_TC, fi), lambda i: (i, 0)),
            pl.BlockSpec((_RB_TC, fi), lambda i: (i, 0)),
            pl.BlockSpec((_RB_TC, 1), lambda i: (i, 0)),
            pl.BlockSpec((fi, fo), lambda i: (0, 0)),
            pl.BlockSpec((1, fo), lambda i: (0, 0)),
        ],
        out_specs=pl.BlockSpec((_RB_TC, fo), lambda i: (i, 0)),
        out_shape=jax.ShapeDtypeStruct((N, fo), jnp.float32),
    )(acc, y, dinv, W, b.reshape(1, fo))


def _counts(batch2d):
    def body(b_ref, o_ref):
        i = pl.program_id(0)
        seg = lax.broadcasted_iota(jnp.int32, (1, G), 1)
        oh = (b_ref[...] == seg).astype(jnp.float32)
        part = jnp.sum(oh, axis=0, keepdims=True)
        @pl.when(i == 0)
        def _():
            o_ref[...] = jnp.zeros_like(o_ref)
        o_ref[...] += part

    return pl.pallas_call(
        body,
        grid=(_GRID,),
        in_specs=[pl.BlockSpec((_RB_TC, 1), lambda i: (i, 0))],
        out_specs=pl.BlockSpec((1, G), lambda i: (0, 0)),
        out_shape=jax.ShapeDtypeStruct((1, G), jnp.float32),
    )(batch2d)


def _stage_d(pmaxs, psums, cnt, Wc, bc):
    F = 64

    def body(pm_ref, ps_ref, cnt_ref, wc_ref, bc_ref, o_ref):
        pm = jnp.max(pm_ref[...].reshape(NC * NS, G, F), axis=0)
        ps = jnp.sum(ps_ref[...].reshape(NC * NS, G, F), axis=0)
        mean = ps / jnp.maximum(cnt_ref[...], 1.0)
        comb = jnp.concatenate([pm, mean], axis=1)
        logits = jnp.dot(comb, wc_ref[...], preferred_element_type=jnp.float32)
        logits = logits + bc_ref[...]
        o_ref[...] = jax.nn.softmax(logits, axis=1)

    return pl.pallas_call(
        body,
        out_shape=jax.ShapeDtypeStruct((G, 2), jnp.float32),
    )(pmaxs, psums, cnt, Wc, bc.reshape(1, 2))


def kernel(x, edge_index, batch, W1, b1, W2, b2, W3, b3, Wc, bc):
    src = edge_index[0]
    dst = edge_index[1]

    degp = _deg(dst).reshape(NC * NS, N_PAD).T

    xp = jnp.pad(x, ((0, 0), (0, 5)))
    W1p = jnp.pad(W1, ((0, 5), (0, 0)))

    dinv, y0 = _stage_a(degp, xp)

    acc1 = _prop(src, dst, y0, F=8, C=50000, RB=4)
    y1 = _stage_b(acc1, y0, dinv, W1p, b1, relu=True, scale=True)

    def _prop2(yv):
        q02 = _prop(src, dst, yv, F=32, C=25000, RB=4, splits=2, loff=0)
        q13 = _prop(src, dst, yv, F=32, C=25000, RB=4, splits=2, loff=1)
        return jnp.concatenate([q02[:25000], q13[:25000],
                                q02[25000:], q13[25000:]], axis=0)

    acc2 = _prop2(y1)
    y2 = _stage_b(acc2, y1, dinv, W2, b2, relu=True, scale=True)

    y2a = y2[:, :32]
    y2b = y2[:, 32:]
    acc3a = _prop2(y2a)
    acc3b = _prop2(y2b)
    h3 = _stage_b3(acc3a, acc3b, y2a, y2b, dinv, W3[:32], W3[32:], b3)

    pmaxs, psums = _pool(h3.reshape(-1), jnp.pad(batch, (0, N_PAD - N)))
    cnt = _counts(batch.reshape(N, 1))

    return _stage_d(pmaxs.reshape(NC * NS, G * 64), psums.reshape(NC * NS, G * 64),
                    cnt.reshape(G, 1), Wc, bc)


# speedup vs baseline: 4.8666x; 4.8666x over previous
import functools

import jax
import jax.numpy as jnp
from jax import lax
from jax.experimental import pallas as pl
from jax.experimental.pallas import tpu as pltpu
from jax.experimental.pallas import tpu_sc as plsc

NC = 2
NS = 16
LN = 16

N = 100000
E = 3200000
G = 128

N_PAD = 100096
STRIPE = N_PAD // NS

ECH = 2000
BLK = 128

_mesh = plsc.VectorSubcoreMesh(core_axis_name="c", subcore_axis_name="s")
_sc_params = pltpu.CompilerParams(needs_layout_passes=False,
                                  use_tc_tiling_on_sc=False)


def _f32(x):
    return jnp.float32(x)


def _deg(dst):
    eslice = E // (NC * NS)
    nch = eslice // ECH

    @functools.partial(
        pl.kernel,
        out_type=jax.ShapeDtypeStruct((NC * NS * N_PAD,), jnp.float32),
        mesh=_mesh,
        compiler_params=_sc_params,
        scratch_types=[
            pltpu.VMEM((N_PAD,), jnp.float32),
            pltpu.VMEM((ECH,), jnp.int32),
        ],
    )
    def k(dst_hbm, out_hbm, dcnt, stage):
        c = lax.axis_index("c")
        s = lax.axis_index("s")
        w = c * NS + s
        zv = jnp.zeros((LN,), jnp.float32)
        ones = jnp.ones((LN,), jnp.float32)

        def zero_body(i, _):
            dcnt[pl.ds(i * LN, LN)] = zv
            return 0
        lax.fori_loop(0, N_PAD // LN, zero_body, 0)

        def chunk(ci, _):
            pltpu.sync_copy(dst_hbm.at[pl.ds(w * eslice + ci * ECH, ECH)], stage)

            def grp(g, _):
                dv = stage[pl.ds(g * LN, LN)]
                plsc.addupdate_scatter(dcnt, [dv], ones)
                return 0
            lax.fori_loop(0, ECH // LN, grp, 0)
            return 0
        lax.fori_loop(0, nch, chunk, 0)

        pltpu.sync_copy(dcnt, out_hbm.at[pl.ds(w * N_PAD, N_PAD)])

    return k(dst)


def _prop(src, dst, y, F, C, RB):
    eslice = E // NS
    nch = eslice // ECH
    cap_rows = (ECH + BLK - 1) // BLK + 1
    ZB = 2048
    czb = ZB * ((C + 8 + ZB - 1) // ZB)
    nzb = czb // ZB
    nwb = C // BLK
    wtail = C - nwb * BLK

    zrows = jnp.zeros((ZB, F), jnp.float32)

    @functools.partial(
        pl.kernel,
        out_type=jax.ShapeDtypeStruct((N, F), jnp.float32),
        mesh=_mesh,
        compiler_params=_sc_params,
        scratch_types=[
            pltpu.VMEM((ECH,), jnp.int32),
            pltpu.VMEM((ECH,), jnp.int32),
            pltpu.VMEM((cap_rows, BLK), jnp.int32),
            pltpu.VMEM((cap_rows, BLK), jnp.int32),
            pltpu.VMEM((RB, BLK, F), jnp.float32),
            pltpu.VMEM_SHARED((czb, F), jnp.float32),
            pltpu.SemaphoreType.DMA((RB,)),
            pltpu.SemaphoreType.DMA((RB,)),
        ],
    )
    def k(src_hbm, dst_hbm, y_hbm, z_hbm, out_hbm,
          ssrc, sdst, isrc, idst, rowbuf, acc, gsem, ssem):
        c = lax.axis_index("c")
        s = lax.axis_index("s")
        iota = lax.iota(jnp.int32, LN)
        trash = jnp.full((LN,), C, jnp.int32)
        zsrc = jnp.zeros((LN,), jnp.int32)

        if True:
            base = c * C

            def zblk(j, _):
                pltpu.sync_copy(z_hbm, acc.at[pl.ds((s + j * NS) * ZB, ZB), :])
                return 0
            lax.fori_loop(0, (nzb - s + NS - 1) // NS, zblk, 0)
            plsc.subcore_barrier()

            def chunk(ci, _):
                off = s * eslice + ci * ECH
                pltpu.sync_copy(src_hbm.at[pl.ds(off, ECH)], ssrc)
                pltpu.sync_copy(dst_hbm.at[pl.ds(off, ECH)], sdst)

                basev = jnp.full((LN,), base, jnp.int32)
                climv = jnp.full((LN,), base + C, jnp.int32)
                c7 = jnp.full((LN,), 7, jnp.int32)
                c127 = jnp.full((LN,), 127, jnp.int32)

                def grp(g, cnt):
                    dv = sdst[pl.ds(g * LN, LN)]
                    sv = ssrc[pl.ds(g * LN, LN)]
                    m = (dv >= basev) & (dv < climv)
                    mi = m.astype(jnp.int32)
                    csum = plsc.cumsum(mi)
                    pos = (jnp.full((LN,), cnt, jnp.int32) + csum) - mi
                    row = lax.shift_right_logical(pos, c7)
                    col = pos & c127
                    plsc.store_scatter(isrc, [row, col], sv, mask=m)
                    plsc.store_scatter(idst, [row, col], dv - basev, mask=m)
                    return cnt + jnp.sum(mi)
                cnt = lax.fori_loop(0, ECH // LN, grp, jnp.int32(0))

                npad = (BLK - (cnt & (BLK - 1))) & (BLK - 1)
                for kk in range(BLK // LN):
                    @pl.when(kk * LN < npad)
                    def _():
                        pos = jnp.full((LN,), cnt + kk * LN, jnp.int32) + iota
                        row = lax.shift_right_logical(pos, c7)
                        col = pos & c127
                        plsc.store_scatter(isrc, [row, col], zsrc)
                        plsc.store_scatter(idst, [row, col], trash)
                nblk = lax.shift_right_logical(cnt + npad, 7)
                nwave = (nblk + RB - 1) // RB

                def wave(gw, _):
                    for b in range(RB):
                        j = gw * RB + b
                        @pl.when(j < nblk)
                        def _():
                            @pl.when(gw > 0)
                            def _():
                                pltpu.make_async_copy(
                                    rowbuf.at[b], acc.at[pl.ds(0, BLK), :],
                                    ssem.at[b]).wait()
                            pltpu.async_copy(
                                y_hbm.at[isrc.at[j]], rowbuf.at[b], gsem.at[b])
                    for b in range(RB):
                        j = gw * RB + b
                        @pl.when(j < nblk)
                        def _():
                            pltpu.make_async_copy(
                                y_hbm.at[isrc.at[j]], rowbuf.at[b],
                                gsem.at[b]).wait()
                            pltpu.async_copy(
                                rowbuf.at[b], acc.at[idst.at[j]],
                                ssem.at[b], add=True)
                    return 0
                lax.fori_loop(0, nwave, wave, 0)

                lws = nblk - (nwave - 1) * RB
                for b in range(RB):
                    @pl.when((nblk > 0) & ((b < lws) | (nwave > 1)))
                    def _():
                        pltpu.make_async_copy(
                            rowbuf.at[b], acc.at[pl.ds(0, BLK), :],
                            ssem.at[b]).wait()
                return 0
            lax.fori_loop(0, nch, chunk, 0)
            plsc.subcore_barrier()

            def wblk(j, _):
                r = (s + j * NS) * BLK
                pltpu.sync_copy(acc.at[pl.ds(r, BLK), :],
                                out_hbm.at[pl.ds(base + r, BLK), :])
                return 0
            lax.fori_loop(0, (nwb - s + NS - 1) // NS, wblk, 0)
            if wtail:
                @pl.when(s == 0)
                def _():
                    pltpu.sync_copy(
                        acc.at[pl.ds(nwb * BLK, wtail), :],
                        out_hbm.at[pl.ds(base + nwb * BLK, wtail), :])
            plsc.subcore_barrier()

    return k(src, dst, y, zrows)


def _pool(h3flat, batchp):
    F = 64
    rslice = N // (NC * NS)
    PCH = 625
    BW = 648
    nch = rslice // PCH
    NG = (PCH + LN - 1) // LN
    FL = F // LN

    @functools.partial(
        pl.kernel,
        out_type=[
            jax.ShapeDtypeStruct((NC * NS * G * F,), jnp.float32),
            jax.ShapeDtypeStruct((NC * NS * G * F,), jnp.float32),
        ],
        mesh=_mesh,
        compiler_params=_sc_params,
        scratch_types=[
            pltpu.VMEM((PCH * F,), jnp.float32),
            pltpu.VMEM((BW,), jnp.int32),
            pltpu.VMEM((G * F,), jnp.float32),
            pltpu.VMEM((G * F,), jnp.float32),
        ],
    )
    def k(h_hbm, b_hbm, omax_hbm, osum_hbm, rows, bvm, pmax, psum):
        c = lax.axis_index("c")
        s = lax.axis_index("s")
        w = c * NS + s
        rbase = w * rslice

        ninf = jnp.full((LN,), -jnp.inf, jnp.float32)
        zv = jnp.zeros((LN,), jnp.float32)
        zi = jnp.zeros((LN,), jnp.int32)
        lane = lax.iota(jnp.int32, LN)

        def init_b(i, _):
            pmax[pl.ds(i * LN, LN)] = ninf
            psum[pl.ds(i * LN, LN)] = zv
            return 0
        lax.fori_loop(0, G * F // LN, init_b, 0)

        def chunk(ci, _):
            start = rbase + ci * PCH
            astart = (start // 8) * 8
            d = start - astart
            pltpu.sync_copy(h_hbm.at[pl.ds(start * F, PCH * F)], rows)
            pltpu.sync_copy(b_hbm.at[pl.ds(astart, BW)], bvm)

            def grpf(g, _):
                bvec = bvm[pl.ds(d + g * LN, LN)]
                for i in range(LN):
                    @pl.when(g * LN + i < PCH)
                    def _():
                        sel = lane == jnp.full((LN,), i, jnp.int32)
                        b_i = jnp.sum(jnp.where(sel, bvec, zi))
                        a = b_i * F
                        r = (g * LN + i) * F
                        for j in range(FL):
                            rv = rows[pl.ds(r + j * LN, LN)]
                            pmax[pl.ds(a + j * LN, LN)] = jnp.maximum(
                                pmax[pl.ds(a + j * LN, LN)], rv)
                            psum[pl.ds(a + j * LN, LN)] = (
                                psum[pl.ds(a + j * LN, LN)] + rv)
                return 0
            lax.fori_loop(0, NG, grpf, 0)
            return 0
        lax.fori_loop(0, nch, chunk, 0)

        pltpu.sync_copy(pmax, omax_hbm.at[pl.ds(w * G * F, G * F)])
        pltpu.sync_copy(psum, osum_hbm.at[pl.ds(w * G * F, G * F)])

    return k(h3flat, batchp)


_RB_TC = 4000
_GRID = N // _RB_TC


def _stage_a(degp, xp):
    def body(dp_ref, xp_ref, dinv_ref, y0_ref):
        d = jnp.sum(dp_ref[...], axis=1, keepdims=True)
        dv = lax.rsqrt(d + 1.0)
        dinv_ref[...] = dv
        y0_ref[...] = xp_ref[...] * dv

    return pl.pallas_call(
        body,
        grid=(_GRID,),
        in_specs=[
            pl.BlockSpec((_RB_TC, NC * NS), lambda i: (i, 0)),
            pl.BlockSpec((_RB_TC, 8), lambda i: (i, 0)),
        ],
        out_specs=[
            pl.BlockSpec((_RB_TC, 1), lambda i: (i, 0)),
            pl.BlockSpec((_RB_TC, 8), lambda i: (i, 0)),
        ],
        out_shape=[
            jax.ShapeDtypeStruct((N, 1), jnp.float32),
            jax.ShapeDtypeStruct((N, 8), jnp.float32),
        ],
    )(degp, xp)


def _stage_b3(acc_a, acc_b, y_a, y_b, dinv, Wa, Wb, b):
    def body(aa_ref, ab_ref, ya_ref, yb_ref, dinv_ref, wa_ref, wb_ref, b_ref, o_ref):
        dv = dinv_ref[...]
        pa = (aa_ref[...] + ya_ref[...]) * dv
        pb = (ab_ref[...] + yb_ref[...]) * dv
        h = (jnp.dot(pa, wa_ref[...], preferred_element_type=jnp.float32)
             + jnp.dot(pb, wb_ref[...], preferred_element_type=jnp.float32)
             + b_ref[...])
        o_ref[...] = h

    return pl.pallas_call(
        body,
        grid=(_GRID,),
        in_specs=[
            pl.BlockSpec((_RB_TC, 32), lambda i: (i, 0)),
            pl.BlockSpec((_RB_TC, 32), lambda i: (i, 0)),
            pl.BlockSpec((_RB_TC, 32), lambda i: (i, 0)),
            pl.BlockSpec((_RB_TC, 32), lambda i: (i, 0)),
            pl.BlockSpec((_RB_TC, 1), lambda i: (i, 0)),
            pl.BlockSpec((32, 64), lambda i: (0, 0)),
            pl.BlockSpec((32, 64), lambda i: (0, 0)),
            pl.BlockSpec((1, 64), lambda i: (0, 0)),
        ],
        out_specs=pl.BlockSpec((_RB_TC, 64), lambda i: (i, 0)),
        out_shape=jax.ShapeDtypeStruct((N, 64), jnp.float32),
    )(acc_a, acc_b, y_a, y_b, dinv, Wa, Wb, b.reshape(1, 64))


def _stage_b(acc, y, dinv, W, b, relu, scale):
    fi, fo = W.shape

    def body(acc_ref, y_ref, dinv_ref, w_ref, b_ref, o_ref):
        p = (acc_ref[...] + y_ref[...]) * dinv_ref[...]
        h = jnp.dot(p, w_ref[...], preferred_element_type=jnp.float32) + b_ref[...]
        if relu:
            h = jnp.maximum(h, 0.0)
        if scale:
            h = h * dinv_ref[...]
        o_ref[...] = h

    return pl.pallas_call(
        body,
        grid=(_GRID,),
        in_specs=[
            pl.BlockSpec((_RB_TC, fi), lambda i: (i, 0)),
            pl.BlockSpec((_RB_TC, fi), lambda i: (i, 0)),
            pl.BlockSpec((_RB_TC, 1), lambda i: (i, 0)),
            pl.BlockSpec((fi, fo), lambda i: (0, 0)),
            pl.BlockSpec((1, fo), lambda i: (0, 0)),
        ],
        out_specs=pl.BlockSpec((_RB_TC, fo), lambda i: (i, 0)),
        out_shape=jax.ShapeDtypeStruct((N, fo), jnp.float32),
    )(acc, y, dinv, W, b.reshape(1, fo))


def _counts(batch2d):
    def body(b_ref, o_ref):
        i = pl.program_id(0)
        seg = lax.broadcasted_iota(jnp.int32, (1, G), 1)
        oh = (b_ref[...] == seg).astype(jnp.float32)
        part = jnp.sum(oh, axis=0, keepdims=True)
        @pl.when(i == 0)
        def _():
            o_ref[...] = jnp.zeros_like(o_ref)
        o_ref[...] += part

    return pl.pallas_call(
        body,
        grid=(_GRID,),
        in_specs=[pl.BlockSpec((_RB_TC, 1), lambda i: (i, 0))],
        out_specs=pl.BlockSpec((1, G), lambda i: (0, 0)),
        out_shape=jax.ShapeDtypeStruct((1, G), jnp.float32),
    )(batch2d)


def _stage_d(pmaxs, psums, cnt, Wc, bc):
    F = 64

    def body(pm_ref, ps_ref, cnt_ref, wc_ref, bc_ref, o_ref):
        pm = jnp.max(pm_ref[...].reshape(NC * NS, G, F), axis=0)
        ps = jnp.sum(ps_ref[...].reshape(NC * NS, G, F), axis=0)
        mean = ps / jnp.maximum(cnt_ref[...], 1.0)
        comb = jnp.concatenate([pm, mean], axis=1)
        logits = jnp.dot(comb, wc_ref[...], preferred_element_type=jnp.float32)
        logits = logits + bc_ref[...]
        o_ref[...] = jax.nn.softmax(logits, axis=1)

    return pl.pallas_call(
        body,
        out_shape=jax.ShapeDtypeStruct((G, 2), jnp.float32),
    )(pmaxs, psums, cnt, Wc, bc.reshape(1, 2))


def kernel(x, edge_index, batch, W1, b1, W2, b2, W3, b3, Wc, bc):
    src = edge_index[0]
    dst = edge_index[1]

    degp = _deg(dst).reshape(NC * NS, N_PAD).T

    xp = jnp.pad(x, ((0, 0), (0, 5)))
    W1p = jnp.pad(W1, ((0, 5), (0, 0)))

    dinv, y0 = _stage_a(degp, xp)

    acc1 = _prop(src, dst, y0, F=8, C=50000, RB=8)
    y1 = _stage_b(acc1, y0, dinv, W1p, b1, relu=True, scale=True)

    acc2 = _prop(src, dst, y1, F=32, C=50000, RB=4)
    y2 = _stage_b(acc2, y1, dinv, W2, b2, relu=True, scale=True)

    y2a = y2[:, :32]
    y2b = y2[:, 32:]
    acc3a = _prop(src, dst, y2a, F=32, C=50000, RB=4)
    acc3b = _prop(src, dst, y2b, F=32, C=50000, RB=4)
    h3 = _stage_b3(acc3a, acc3b, y2a, y2b, dinv, W3[:32], W3[32:], b3)

    pmaxs, psums = _pool(h3.reshape(-1), jnp.pad(batch, (0, N_PAD - N)))
    cnt = _counts(batch.reshape(N, 1))

    return _stage_d(pmaxs.reshape(NC * NS, G * 64), psums.reshape(NC * NS, G * 64),
                    cnt.reshape(G, 1), Wc, bc)
